# baseline (speedup 1.0000x reference)
import functools

import jax
import jax.numpy as jnp
from jax import lax
from jax.experimental import pallas as pl
from jax.experimental.pallas import tpu as pltpu
from jax.experimental.pallas import tpu_sc as plsc

SEQ = 32768
NE = 64
NG = 8
GS = NE // NG
NC, NS, L = 2, 16, 16
NW = NC * NS
ROWS_PER_W = SEQ // NW
CHUNK = 256
NCHUNK = ROWS_PER_W // CHUNK
NBLK = CHUNK // L


def _router_body(logits_hbm, rw_hbm, w_hbm, ids_hbm, cnt_hbm,
                 in_v, rw_v, w_v, ids_v, cnt_v):
    wid = lax.axis_index("s") * NC + lax.axis_index("c")
    base = wid * ROWS_PER_W

    lanes = jnp.arange(L, dtype=jnp.int32)
    zeros_f = jnp.zeros((L,), jnp.float32)
    ones_f = jnp.ones((L,), jnp.float32)

    for e in range(NE):
        cnt_v[pl.ds(e * L, L)] = zeros_f

    def block_body(b, _):
        addr = (b * L + lanes) * NE
        addr8 = (b * L + lanes) * NG

        gmax = []
        gidx = []
        for g in range(NG):
            best = plsc.load_gather(in_v, [addr + (g * GS)])
            bidx = jnp.zeros((L,), jnp.int32)
            for j in range(1, GS):
                x = plsc.load_gather(in_v, [addr + (g * GS + j)])
                cgt = x > best
                best = jnp.where(cgt, x, best)
                bidx = jnp.where(cgt, jnp.full((L,), j, jnp.int32), bidx)
            gmax.append(best)
            gidx.append(bidx)

        m = gmax[0]
        for g in range(1, NG):
            m = jnp.maximum(m, gmax[g])

        pg = [jnp.exp(gmax[g] - m) for g in range(NG)]
        tot = pg[0]
        for g in range(1, NG):
            tot = tot + pg[g]
        tinv = ones_f / tot
        for g in range(NG):
            plsc.store_scatter(w_v, [addr8 + g], pg[g] * tinv)
            gid = gidx[g] + (g * GS)
            plsc.store_scatter(ids_v, [addr8 + g], gid)
            plsc.addupdate_scatter(cnt_v, [gid * L + lanes], ones_f)

        s = jnp.zeros((L,), jnp.float32)
        for e in range(NE):
            x = plsc.load_gather(in_v, [addr + e])
            s = s + jnp.exp(x - m)
        sinv = ones_f / s

        for e in range(NE):
            x = plsc.load_gather(in_v, [addr + e])
            plsc.store_scatter(rw_v, [addr + e], jnp.exp(x - m) * sinv)
        return 0

    for c in range(NCHUNK):
        row0 = base + c * CHUNK
        pltpu.sync_copy(logits_hbm.at[pl.ds(row0 * NE, CHUNK * NE)], in_v)
        lax.fori_loop(0, NBLK, block_body, 0)
        pltpu.sync_copy(rw_v, rw_hbm.at[pl.ds(row0 * NE, CHUNK * NE)])
        pltpu.sync_copy(w_v, w_hbm.at[pl.ds(row0 * NG, CHUNK * NG)])
        pltpu.sync_copy(ids_v, ids_hbm.at[pl.ds(row0 * NG, CHUNK * NG)])

    acc = [jnp.zeros((L,), jnp.float32) for _ in range(NE // L)]
    for e in range(NE):
        v = cnt_v[pl.ds(e * L, L)]
        sv = jnp.full((L,), jnp.sum(v), jnp.float32)
        q, r = divmod(e, L)
        acc[q] = jnp.where(lanes == r, sv, acc[q])
    for q in range(NE // L):
        cnt_v[pl.ds(q * L, L)] = acc[q]
    pltpu.sync_copy(cnt_v.at[pl.ds(0, NE)], cnt_hbm.at[pl.ds(wid * NE, NE)])


_router = functools.partial(
    pl.kernel,
    out_type=[
        jax.ShapeDtypeStruct((SEQ * NE,), jnp.float32),
        jax.ShapeDtypeStruct((SEQ * NG,), jnp.float32),
        jax.ShapeDtypeStruct((SEQ * NG,), jnp.int32),
        jax.ShapeDtypeStruct((NW * NE,), jnp.float32),
    ],
    mesh=plsc.VectorSubcoreMesh(core_axis_name="c", subcore_axis_name="s",
                                num_cores=NC, num_subcores=NS),
    compiler_params=pltpu.CompilerParams(needs_layout_passes=False),
    scratch_types=[
        pltpu.VMEM((CHUNK * NE,), jnp.float32),
        pltpu.VMEM((CHUNK * NE,), jnp.float32),
        pltpu.VMEM((CHUNK * NG,), jnp.float32),
        pltpu.VMEM((CHUNK * NG,), jnp.int32),
        pltpu.VMEM((NE * L,), jnp.float32),
    ],
)(_router_body)


@jax.jit
def kernel(logits):
    rw_f, w_f, ids_f, cnt_part = _router(logits.reshape(-1))
    routing_weights = rw_f.reshape(SEQ, NE)
    topk_weights = w_f.reshape(SEQ, NG)
    topk_ids = ids_f.reshape(SEQ, NG)
    tokens_per_expert = cnt_part.reshape(NW, NE).sum(axis=0)
    return (logits, routing_weights, topk_weights, topk_ids, tokens_per_expert)


# speedup vs baseline: 4.0712x; 4.0712x over previous
import functools

import jax
import jax.numpy as jnp
from jax import lax
from jax.experimental import pallas as pl
from jax.experimental.pallas import tpu as pltpu
from jax.experimental.pallas import tpu_sc as plsc

SEQ = 32768
NE = 64
NG = 8
GS = NE // NG
NC, NS, L = 2, 16, 16
NW = NC * NS
ROWS_PER_W = SEQ // NW
CR = 512
NCHUNK = ROWS_PER_W // CR
NBLK = CR // L


def _router_body(in_hbm, rw_hbm, w_hbm, ids_hbm, cnt_hbm,
                 in_v, rw_v, w_v, ids_v, cnt_v):
    wid = lax.axis_index("s") * NC + lax.axis_index("c")
    base = wid * ROWS_PER_W

    lanes = jnp.arange(L, dtype=jnp.int32)
    zeros_f = jnp.zeros((L,), jnp.float32)
    ones_f = jnp.ones((L,), jnp.float32)

    for e in range(NE):
        cnt_v[pl.ds(e * L, L)] = zeros_f

    def block_body(b, _):
        r = b * L

        gmax = []
        gidx = []
        for g in range(NG):
            best = in_v[g * GS, pl.ds(r, L)]
            bidx = jnp.zeros((L,), jnp.int32)
            for j in range(1, GS):
                x = in_v[g * GS + j, pl.ds(r, L)]
                cgt = x > best
                best = jnp.where(cgt, x, best)
                bidx = jnp.where(cgt, jnp.full((L,), j, jnp.int32), bidx)
            gmax.append(best)
            gidx.append(bidx)

        m01 = jnp.maximum(gmax[0], gmax[1])
        m23 = jnp.maximum(gmax[2], gmax[3])
        m45 = jnp.maximum(gmax[4], gmax[5])
        m67 = jnp.maximum(gmax[6], gmax[7])
        m = jnp.maximum(jnp.maximum(m01, m23), jnp.maximum(m45, m67))

        pg = [jnp.exp(gmax[g] - m) for g in range(NG)]
        tot = ((pg[0] + pg[1]) + (pg[2] + pg[3])) + \
              ((pg[4] + pg[5]) + (pg[6] + pg[7]))
        tinv = ones_f / tot
        for g in range(NG):
            w_v[g, pl.ds(r, L)] = pg[g] * tinv
            gid = gidx[g] + (g * GS)
            ids_v[g, pl.ds(r, L)] = gid
            plsc.addupdate_scatter(cnt_v, [gid * L + lanes], ones_f)

        s0 = zeros_f
        s1 = zeros_f
        s2 = zeros_f
        s3 = zeros_f
        for e in range(0, NE, 4):
            p0 = jnp.exp(in_v[e, pl.ds(r, L)] - m)
            p1 = jnp.exp(in_v[e + 1, pl.ds(r, L)] - m)
            p2 = jnp.exp(in_v[e + 2, pl.ds(r, L)] - m)
            p3 = jnp.exp(in_v[e + 3, pl.ds(r, L)] - m)
            rw_v[e, pl.ds(r, L)] = p0
            rw_v[e + 1, pl.ds(r, L)] = p1
            rw_v[e + 2, pl.ds(r, L)] = p2
            rw_v[e + 3, pl.ds(r, L)] = p3
            s0 = s0 + p0
            s1 = s1 + p1
            s2 = s2 + p2
            s3 = s3 + p3
        sinv = ones_f / ((s0 + s1) + (s2 + s3))

        for e in range(NE):
            rw_v[e, pl.ds(r, L)] = rw_v[e, pl.ds(r, L)] * sinv
        return 0

    for c in range(NCHUNK):
        row0 = base + c * CR
        pltpu.sync_copy(in_hbm.at[:, pl.ds(row0, CR)], in_v)
        lax.fori_loop(0, NBLK, block_body, 0)
        pltpu.sync_copy(rw_v, rw_hbm.at[:, pl.ds(row0, CR)])
        pltpu.sync_copy(w_v, w_hbm.at[:, pl.ds(row0, CR)])
        pltpu.sync_copy(ids_v, ids_hbm.at[:, pl.ds(row0, CR)])

    acc = [jnp.zeros((L,), jnp.float32) for _ in range(NE // L)]
    for e in range(NE):
        v = cnt_v[pl.ds(e * L, L)]
        sv = jnp.full((L,), jnp.sum(v), jnp.float32)
        q, rr = divmod(e, L)
        acc[q] = jnp.where(lanes == rr, sv, acc[q])
    for q in range(NE // L):
        cnt_v[pl.ds(q * L, L)] = acc[q]
    pltpu.sync_copy(cnt_v.at[pl.ds(0, NE)], cnt_hbm.at[pl.ds(wid * NE, NE)])


_router = functools.partial(
    pl.kernel,
    out_type=[
        jax.ShapeDtypeStruct((NE, SEQ), jnp.float32),
        jax.ShapeDtypeStruct((NG, SEQ), jnp.float32),
        jax.ShapeDtypeStruct((NG, SEQ), jnp.int32),
        jax.ShapeDtypeStruct((NW * NE,), jnp.float32),
    ],
    mesh=plsc.VectorSubcoreMesh(core_axis_name="c", subcore_axis_name="s",
                                num_cores=NC, num_subcores=NS),
    compiler_params=pltpu.CompilerParams(needs_layout_passes=False),
    scratch_types=[
        pltpu.VMEM((NE, CR), jnp.float32),
        pltpu.VMEM((NE, CR), jnp.float32),
        pltpu.VMEM((NG, CR), jnp.float32),
        pltpu.VMEM((NG, CR), jnp.int32),
        pltpu.VMEM((NE * L,), jnp.float32),
    ],
)(_router_body)


@jax.jit
def kernel(logits):
    rw_t, w_t, ids_t, cnt_part = _router(logits.T)
    routing_weights = rw_t.T
    topk_weights = w_t.T
    topk_ids = ids_t.T
    tokens_per_expert = cnt_part.reshape(NW, NE).sum(axis=0)
    return (logits, routing_weights, topk_weights, topk_ids, tokens_per_expert)


# speedup vs baseline: 4.5817x; 1.1254x over previous
import functools

import jax
import jax.numpy as jnp
from jax import lax
from jax.experimental import pallas as pl
from jax.experimental.pallas import tpu as pltpu
from jax.experimental.pallas import tpu_sc as plsc

SEQ = 32768
NE = 64
NG = 8
GS = NE // NG
NC, NS, L = 2, 16, 16
NW = NC * NS
ROWS_PER_W = SEQ // NW
CR = 512
NCHUNK = ROWS_PER_W // CR
NBLK = CR // L


def _router_body(in_hbm, rw_hbm, w_hbm, ids_hbm, cnt_hbm,
                 in_v, rw_v, w_v, ids_v, cnt_v):
    wid = lax.axis_index("s") * NC + lax.axis_index("c")
    base = wid * ROWS_PER_W

    lanes = jnp.arange(L, dtype=jnp.int32)
    zeros_f = jnp.zeros((L,), jnp.float32)
    ones_f = jnp.ones((L,), jnp.float32)

    for e in range(NE):
        cnt_v[pl.ds(e * L, L)] = zeros_f

    def block_body(b):
        r = b * L

        gmax = []
        gidx = []
        tg = []
        for g in range(NG):
            x = [in_v[g * GS + j, pl.ds(r, L)] for j in range(GS)]
            best = x[0]
            bidx = jnp.zeros((L,), jnp.int32)
            for j in range(1, GS):
                cgt = x[j] > best
                best = jnp.where(cgt, x[j], best)
                bidx = jnp.where(cgt, jnp.full((L,), j, jnp.int32), bidx)
            q = [jnp.exp(x[j] - best) for j in range(GS)]
            for j in range(GS):
                rw_v[g * GS + j, pl.ds(r, L)] = q[j]
            t = ((q[0] + q[1]) + (q[2] + q[3])) + \
                ((q[4] + q[5]) + (q[6] + q[7]))
            gmax.append(best)
            gidx.append(bidx)
            tg.append(t)

        m = jnp.maximum(
            jnp.maximum(jnp.maximum(gmax[0], gmax[1]),
                        jnp.maximum(gmax[2], gmax[3])),
            jnp.maximum(jnp.maximum(gmax[4], gmax[5]),
                        jnp.maximum(gmax[6], gmax[7])))

        pg = [jnp.exp(gmax[g] - m) for g in range(NG)]
        tot = ((pg[0] + pg[1]) + (pg[2] + pg[3])) + \
              ((pg[4] + pg[5]) + (pg[6] + pg[7]))
        tinv = ones_f / tot
        st = [pg[g] * tg[g] for g in range(NG)]
        s = ((st[0] + st[1]) + (st[2] + st[3])) + \
            ((st[4] + st[5]) + (st[6] + st[7]))
        sinv = ones_f / s

        for g in range(NG):
            w_v[g, pl.ds(r, L)] = pg[g] * tinv
            gid = gidx[g] + (g * GS)
            ids_v[g, pl.ds(r, L)] = gid
            plsc.addupdate_scatter(cnt_v, [gid * L + lanes], ones_f)
            fct = pg[g] * sinv
            for j in range(GS):
                e = g * GS + j
                rw_v[e, pl.ds(r, L)] = rw_v[e, pl.ds(r, L)] * fct

    for c in range(NCHUNK):
        row0 = base + c * CR
        pltpu.sync_copy(in_hbm.at[:, pl.ds(row0, CR)], in_v)
        plsc.parallel_loop(0, NBLK, 1, unroll=2)(block_body)
        pltpu.sync_copy(rw_v, rw_hbm.at[:, pl.ds(row0, CR)])
        pltpu.sync_copy(w_v, w_hbm.at[:, pl.ds(row0, CR)])
        pltpu.sync_copy(ids_v, ids_hbm.at[:, pl.ds(row0, CR)])

    acc = [jnp.zeros((L,), jnp.float32) for _ in range(NE // L)]
    for e in range(NE):
        v = cnt_v[pl.ds(e * L, L)]
        sv = jnp.full((L,), jnp.sum(v), jnp.float32)
        q, rr = divmod(e, L)
        acc[q] = jnp.where(lanes == rr, sv, acc[q])
    for q in range(NE // L):
        cnt_v[pl.ds(q * L, L)] = acc[q]
    pltpu.sync_copy(cnt_v.at[pl.ds(0, NE)], cnt_hbm.at[pl.ds(wid * NE, NE)])


_router = functools.partial(
    pl.kernel,
    out_type=[
        jax.ShapeDtypeStruct((NE, SEQ), jnp.float32),
        jax.ShapeDtypeStruct((NG, SEQ), jnp.float32),
        jax.ShapeDtypeStruct((NG, SEQ), jnp.int32),
        jax.ShapeDtypeStruct((NW * NE,), jnp.float32),
    ],
    mesh=plsc.VectorSubcoreMesh(core_axis_name="c", subcore_axis_name="s",
                                num_cores=NC, num_subcores=NS),
    compiler_params=pltpu.CompilerParams(needs_layout_passes=False),
    scratch_types=[
        pltpu.VMEM((NE, CR), jnp.float32),
        pltpu.VMEM((NE, CR), jnp.float32),
        pltpu.VMEM((NG, CR), jnp.float32),
        pltpu.VMEM((NG, CR), jnp.int32),
        pltpu.VMEM((NE * L,), jnp.float32),
    ],
)(_router_body)


@jax.jit
def kernel(logits):
    rw_t, w_t, ids_t, cnt_part = _router(logits.T)
    routing_weights = rw_t.T
    topk_weights = w_t.T
    topk_ids = ids_t.T
    tokens_per_expert = cnt_part.reshape(NW, NE).sum(axis=0)
    return (logits, routing_weights, topk_weights, topk_ids, tokens_per_expert)


# speedup vs baseline: 4.6470x; 1.0142x over previous
import functools

import jax
import jax.numpy as jnp
from jax import lax
from jax.experimental import pallas as pl
from jax.experimental.pallas import tpu as pltpu
from jax.experimental.pallas import tpu_sc as plsc

SEQ = 32768
NE = 64
NG = 8
GS = NE // NG
NC, NS, L = 2, 16, 16
NW = NC * NS
ROWS_PER_W = SEQ // NW
CR = 256
NCHUNK = ROWS_PER_W // CR
NBLK = CR // L


def _treemax(vals):
    while len(vals) > 1:
        vals = [jnp.maximum(vals[2 * i], vals[2 * i + 1])
                for i in range(len(vals) // 2)]
    return vals[0]


def _treemin(vals):
    while len(vals) > 1:
        vals = [jnp.minimum(vals[2 * i], vals[2 * i + 1])
                for i in range(len(vals) // 2)]
    return vals[0]


def _treesum(vals):
    while len(vals) > 1:
        vals = [vals[2 * i] + vals[2 * i + 1]
                for i in range(len(vals) // 2)]
    return vals[0]


def _router_body(in_hbm, rw_hbm, w_hbm, ids_hbm, cnt_hbm,
                 in_v, rw_v, w_v, ids_v, cnt_v,
                 sem_in0, sem_in1, sem_out0, sem_out1):
    sem_in = [sem_in0, sem_in1]
    sem_out = [sem_out0, sem_out1]
    wid = lax.axis_index("s") * NC + lax.axis_index("c")
    base = wid * ROWS_PER_W

    lanes = jnp.arange(L, dtype=jnp.int32)
    zeros_f = jnp.zeros((L,), jnp.float32)
    ones_f = jnp.ones((L,), jnp.float32)

    for e in range(NE):
        cnt_v[pl.ds(e * L, L)] = zeros_f

    def make_block_body(ibuf):
        in_b = in_v.at[ibuf]
        rw_b = rw_v.at[ibuf]
        w_b = w_v.at[ibuf]
        ids_b = ids_v.at[ibuf]

        def block_body(b):
            r = b * L

            gmax = []
            gidx = []
            tg = []
            for g in range(NG):
                x = [in_b[g * GS + j, pl.ds(r, L)] for j in range(GS)]
                best = _treemax(list(x))
                cand = [jnp.where(x[j] == best,
                                  jnp.full((L,), j, jnp.int32),
                                  jnp.full((L,), GS, jnp.int32))
                        for j in range(GS)]
                bidx = _treemin(cand)
                q = [jnp.exp(x[j] - best) for j in range(GS)]
                for j in range(GS):
                    rw_b[g * GS + j, pl.ds(r, L)] = q[j]
                t = _treesum(q)
                gmax.append(best)
                gidx.append(bidx)
                tg.append(t)

            m = _treemax(list(gmax))
            pg = [jnp.exp(gmax[g] - m) for g in range(NG)]
            tot = _treesum(list(pg))
            tinv = ones_f / tot
            s = _treesum([pg[g] * tg[g] for g in range(NG)])
            sinv = ones_f / s

            for g in range(NG):
                w_b[g, pl.ds(r, L)] = pg[g] * tinv
                gid = gidx[g] + (g * GS)
                ids_b[g, pl.ds(r, L)] = gid
                plsc.addupdate_scatter(cnt_v, [gid * L + lanes], ones_f)
                fct = pg[g] * sinv
                for j in range(GS):
                    e = g * GS + j
                    rw_b[e, pl.ds(r, L)] = rw_b[e, pl.ds(r, L)] * fct

        return block_body

    def start_in(c):
        row0 = base + c * CR
        return pltpu.async_copy(in_hbm.at[:, pl.ds(row0, CR)],
                                in_v.at[c % 2], sem_in[c % 2])

    in_dma = [start_in(0)]
    out_dma = {}
    for c in range(NCHUNK):
        if c + 1 < NCHUNK:
            in_dma.append(start_in(c + 1))
        in_dma[c].wait()
        if c >= 2:
            for h in out_dma[c - 2]:
                h.wait()
        plsc.parallel_loop(0, NBLK, 1, unroll=2)(make_block_body(c % 2))
        row0 = base + c * CR
        out_dma[c] = [
            pltpu.async_copy(rw_v.at[c % 2], rw_hbm.at[:, pl.ds(row0, CR)],
                             sem_out[c % 2]),
            pltpu.async_copy(w_v.at[c % 2], w_hbm.at[:, pl.ds(row0, CR)],
                             sem_out[c % 2]),
            pltpu.async_copy(ids_v.at[c % 2], ids_hbm.at[:, pl.ds(row0, CR)],
                             sem_out[c % 2]),
        ]
    for c in range(max(0, NCHUNK - 2), NCHUNK):
        for h in out_dma[c]:
            h.wait()

    acc = [jnp.zeros((L,), jnp.float32) for _ in range(NE // L)]
    for e in range(NE):
        v = cnt_v[pl.ds(e * L, L)]
        sv = jnp.full((L,), jnp.sum(v), jnp.float32)
        q, rr = divmod(e, L)
        acc[q] = jnp.where(lanes == rr, sv, acc[q])
    for q in range(NE // L):
        cnt_v[pl.ds(q * L, L)] = acc[q]
    pltpu.sync_copy(cnt_v.at[pl.ds(0, NE)], cnt_hbm.at[pl.ds(wid * NE, NE)])


_router = functools.partial(
    pl.kernel,
    out_type=[
        jax.ShapeDtypeStruct((NE, SEQ), jnp.float32),
        jax.ShapeDtypeStruct((NG, SEQ), jnp.float32),
        jax.ShapeDtypeStruct((NG, SEQ), jnp.int32),
        jax.ShapeDtypeStruct((NW * NE,), jnp.float32),
    ],
    mesh=plsc.VectorSubcoreMesh(core_axis_name="c", subcore_axis_name="s",
                                num_cores=NC, num_subcores=NS),
    compiler_params=pltpu.CompilerParams(needs_layout_passes=False),
    scratch_types=[
        pltpu.VMEM((2, NE, CR), jnp.float32),
        pltpu.VMEM((2, NE, CR), jnp.float32),
        pltpu.VMEM((2, NG, CR), jnp.float32),
        pltpu.VMEM((2, NG, CR), jnp.int32),
        pltpu.VMEM((NE * L,), jnp.float32),
        pltpu.SemaphoreType.DMA,
        pltpu.SemaphoreType.DMA,
        pltpu.SemaphoreType.DMA,
        pltpu.SemaphoreType.DMA,
    ],
)(_router_body)


@jax.jit
def kernel(logits):
    rw_t, w_t, ids_t, cnt_part = _router(logits.T)
    routing_weights = rw_t.T
    topk_weights = w_t.T
    topk_ids = ids_t.T
    tokens_per_expert = cnt_part.reshape(NW, NE).sum(axis=0)
    return (logits, routing_weights, topk_weights, topk_ids, tokens_per_expert)
